# Initial kernel scaffold; baseline (speedup 1.0000x reference)
import jax
import jax.numpy as jnp
from jax.experimental import pallas as pl


def kernel(x, W1, D1, W2, D2, W3, D3, W4, D4, W5, D5, P1, P2, P3, P4):
    raise NotImplementedError("write your pallas kernel here")


# speedup vs baseline: 1.0750x; 1.0750x over previous
import functools

import jax
import jax.numpy as jnp
from jax.experimental import pallas as pl

EPS = 1e-6
BN_EPS = 1e-5
NEG_SLOPE = 0.2
K = 20
N = 1024
ROWS = 256


def _knn_kernel(coord_ref, coordt_ref, idx_ref):
    c = coord_ref[0]
    ct = coordt_ref[0]
    g = jnp.dot(ct.astype(jnp.bfloat16), c.astype(jnp.bfloat16),
                preferred_element_type=jnp.float32)
    inner2 = -2.0 * g
    xx = jnp.sum(c * c, axis=0, keepdims=True)
    xxr = jnp.sum(ct * ct, axis=1, keepdims=True)
    pd = (-xx - inner2) - xxr
    lane = jax.lax.broadcasted_iota(jnp.int32, (ROWS, N), 1)
    cols = []
    for _ in range(K):
        am = jnp.argmax(pd, axis=1, keepdims=True)
        cols.append(am.astype(jnp.int32))
        pd = jnp.where(lane == am, -jnp.inf, pd)
    cols += [jnp.zeros((ROWS, 1), jnp.int32)] * (32 - K)
    idx_ref[0] = jnp.concatenate(cols, axis=1)


def _knn_topk(coord):
    B = coord.shape[0]
    coordt = jnp.transpose(coord, (0, 2, 1))
    idx = pl.pallas_call(
        _knn_kernel,
        grid=(B, N // ROWS),
        in_specs=[
            pl.BlockSpec((1, 3, N), lambda b, r: (b, 0, 0)),
            pl.BlockSpec((1, ROWS, 3), lambda b, r: (b, r, 0)),
        ],
        out_specs=pl.BlockSpec((1, ROWS, 32), lambda b, r: (b, r, 0)),
        out_shape=jax.ShapeDtypeStruct((B, N, 32), jnp.int32),
    )(coord, coordt)
    return idx[:, :, :K]


def _graph_feature(x, idx):
    B, C, _, Np = x.shape
    xf = jnp.transpose(x.reshape(B, C * 3, Np), (0, 2, 1))
    feat = jax.vmap(lambda xb, ib: xb[ib])(xf, idx)
    feat = feat.reshape(B, Np, K, C, 3)
    xc = jnp.broadcast_to(xf.reshape(B, Np, 1, C, 3), (B, Np, K, C, 3))
    out = jnp.concatenate([feat - xc, xc], axis=3)
    return jnp.transpose(out, (0, 3, 4, 1, 2))


def _vn_bn(p, dim):
    n = jnp.linalg.norm(p, axis=2) + EPS
    axes = (0, 2, 3) if dim == 5 else (0, 2)
    m = jnp.mean(n, axis=axes, keepdims=True)
    v = jnp.mean((n - m) ** 2, axis=axes, keepdims=True)
    nbn = (n - m) / jnp.sqrt(v + BN_EPS)
    return p / jnp.expand_dims(n, 2) * jnp.expand_dims(nbn, 2)


def _vn_lrelu(x, W, D, dim):
    p = jnp.einsum('oi,bi...->bo...', W, x)
    p = _vn_bn(p, dim)
    d = jnp.einsum('oi,bi...->bo...', D, x)
    dot = jnp.sum(p * d, axis=2, keepdims=True)
    dsq = jnp.sum(d * d, axis=2, keepdims=True)
    mask = (dot >= 0).astype(p.dtype)
    relu = mask * p + (1.0 - mask) * (p - (dot / (dsq + EPS)) * d)
    return NEG_SLOPE * p + (1.0 - NEG_SLOPE) * relu


def _vn_maxpool(x, P):
    B, C, _, Np, Kn = x.shape
    d = jnp.einsum('oi,bi...->bo...', P, x)
    dot = jnp.sum(x * d, axis=2, keepdims=True)
    am = jnp.argmax(dot, axis=-1)
    idxb = jnp.broadcast_to(am[..., None], (B, C, 3, Np, 1))
    return jnp.take_along_axis(x, idxb, axis=-1)[..., 0]


def kernel(x, W1, D1, W2, D2, W3, D3, W4, D4, W5, D5, P1, P2, P3, P4):
    coord = x
    idx = _knn_topk(coord)
    h = _vn_lrelu(_graph_feature(x[:, None, :, :], idx), W1, D1, 5)
    x1 = _vn_maxpool(h, P1)
    h = _vn_lrelu(_graph_feature(x1, idx), W2, D2, 5)
    x2 = _vn_maxpool(h, P2)
    h = _vn_lrelu(_graph_feature(x2, idx), W3, D3, 5)
    x3 = _vn_maxpool(h, P3)
    h = _vn_lrelu(_graph_feature(x3, idx), W4, D4, 5)
    x4 = _vn_maxpool(h, P4)
    xp = jnp.concatenate([x1, x2, x3, x4], axis=1)
    return (x4, xp, coord)
